# Initial kernel scaffold; baseline (speedup 1.0000x reference)
import jax
import jax.numpy as jnp
from jax.experimental import pallas as pl


def kernel(x, mod_sig):
    raise NotImplementedError("write your pallas kernel here")


# speedup vs baseline: 742.3328x; 742.3328x over previous
import functools

import jax
import jax.numpy as jnp
from jax import lax
from jax.experimental import pallas as pl
from jax.experimental.pallas import tpu as pltpu
from jax.experimental.pallas import tpu_sc as plsc

_D = 441
_B, _C, _T = 32, 2, 16384
_L = 16
_PAD = 448


def _flanger_body(x_hbm, mod_hbm, out_hbm, x0_v, x1_v, m_v, o0_v, o1_v):
    b = lax.axis_index("s") * 2 + lax.axis_index("c")
    pltpu.sync_copy(x_hbm.at[b, 0], x0_v.at[pl.ds(_PAD, _T)])
    pltpu.sync_copy(x_hbm.at[b, 1], x1_v.at[pl.ds(_PAD, _T)])
    pltpu.sync_copy(mod_hbm.at[b], m_v)

    zeros = jnp.zeros((_L,), jnp.float32)
    for j in range(_PAD // _L):
        x0_v[pl.ds(j * _L, _L)] = zeros
        x1_v[pl.ds(j * _L, _L)] = zeros

    lane = lax.iota(jnp.int32, _L)

    def body(it, _):
        t0 = it * _L
        tv = t0 + lane
        mv = m_v[pl.ds(t0, _L)]
        u448 = tv.astype(jnp.float32) + (jnp.float32(_PAD) - jnp.float32(_D) * mv)
        i448 = u448.astype(jnp.int32)
        frac = u448 - i448.astype(jnp.float32)
        dp = i448 - tv - _PAD
        wp = lax.shift_right_arithmetic(dp, 31)
        gp = i448 - _D - _D * wp
        wn = lax.shift_right_arithmetic(dp + 1, 31)
        gn = i448 + 1 - _D - _D * wn
        pv0 = plsc.load_gather(x0_v, [gp])
        nv0 = plsc.load_gather(x0_v, [gn])
        pv1 = plsc.load_gather(x1_v, [gp])
        nv1 = plsc.load_gather(x1_v, [gn])
        omf = 1.0 - frac
        o0_v[pl.ds(t0, _L)] = x0_v[pl.ds(t0 + _PAD, _L)] + frac * nv0 + omf * pv0
        o1_v[pl.ds(t0, _L)] = x1_v[pl.ds(t0 + _PAD, _L)] + frac * nv1 + omf * pv1
        return 0

    lax.fori_loop(0, _T // _L, body, 0)

    pltpu.sync_copy(o0_v, out_hbm.at[b, 0])
    pltpu.sync_copy(o1_v, out_hbm.at[b, 1])


@jax.jit
def _flanger(x, mod_sig):
    mesh = plsc.VectorSubcoreMesh(core_axis_name="c", subcore_axis_name="s")
    fn = functools.partial(
        pl.kernel,
        mesh=mesh,
        compiler_params=pltpu.CompilerParams(
            needs_layout_passes=False, use_tc_tiling_on_sc=False
        ),
        out_type=jax.ShapeDtypeStruct((_B, _C, _T), jnp.float32),
        scratch_types=[
            pltpu.VMEM((_PAD + _T,), jnp.float32),
            pltpu.VMEM((_PAD + _T,), jnp.float32),
            pltpu.VMEM((_T,), jnp.float32),
            pltpu.VMEM((_T,), jnp.float32),
            pltpu.VMEM((_T,), jnp.float32),
        ],
    )(_flanger_body)
    return fn(x, mod_sig)


def kernel(x, mod_sig):
    return _flanger(x, mod_sig)


# speedup vs baseline: 1164.5451x; 1.5688x over previous
import functools

import jax
import jax.numpy as jnp
from jax import lax
from jax.experimental import pallas as pl
from jax.experimental.pallas import tpu as pltpu
from jax.experimental.pallas import tpu_sc as plsc

_D = 441
_B, _C, _T = 32, 2, 16384
_L = 16
_PAD = 448


def _flanger_body(x_hbm, mod_hbm, out_hbm, x0_v, x1_v, m_v, o0_v, o1_v):
    b = lax.axis_index("s") * 2 + lax.axis_index("c")
    pltpu.sync_copy(x_hbm.at[b, 0], x0_v.at[pl.ds(_PAD, _T)])
    pltpu.sync_copy(x_hbm.at[b, 1], x1_v.at[pl.ds(_PAD, _T)])
    pltpu.sync_copy(mod_hbm.at[b], m_v)

    zeros = jnp.zeros((_L,), jnp.float32)
    for j in range(_PAD // _L):
        x0_v[pl.ds(j * _L, _L)] = zeros
        x1_v[pl.ds(j * _L, _L)] = zeros

    lane = lax.iota(jnp.int32, _L)

    @plsc.parallel_loop(0, _T // _L, unroll=8)
    def body(it):
        t0 = it * _L
        tv = t0 + lane
        mv = m_v[pl.ds(t0, _L)]
        u448 = tv.astype(jnp.float32) + (jnp.float32(_PAD) - jnp.float32(_D) * mv)
        i448 = u448.astype(jnp.int32)
        frac = u448 - i448.astype(jnp.float32)
        dp = i448 - tv - _PAD
        wp = lax.shift_right_arithmetic(dp, 31)
        gp = i448 - _D - _D * wp
        wn = lax.shift_right_arithmetic(dp + 1, 31)
        gn = i448 + 1 - _D - _D * wn
        pv0 = plsc.load_gather(x0_v, [gp])
        nv0 = plsc.load_gather(x0_v, [gn])
        pv1 = plsc.load_gather(x1_v, [gp])
        nv1 = plsc.load_gather(x1_v, [gn])
        omf = 1.0 - frac
        o0_v[pl.ds(t0, _L)] = x0_v[pl.ds(t0 + _PAD, _L)] + frac * nv0 + omf * pv0
        o1_v[pl.ds(t0, _L)] = x1_v[pl.ds(t0 + _PAD, _L)] + frac * nv1 + omf * pv1

    pltpu.sync_copy(o0_v, out_hbm.at[b, 0])
    pltpu.sync_copy(o1_v, out_hbm.at[b, 1])


@jax.jit
def _flanger(x, mod_sig):
    mesh = plsc.VectorSubcoreMesh(core_axis_name="c", subcore_axis_name="s")
    fn = functools.partial(
        pl.kernel,
        mesh=mesh,
        compiler_params=pltpu.CompilerParams(
            needs_layout_passes=False, use_tc_tiling_on_sc=False
        ),
        out_type=jax.ShapeDtypeStruct((_B, _C, _T), jnp.float32),
        scratch_types=[
            pltpu.VMEM((_PAD + _T,), jnp.float32),
            pltpu.VMEM((_PAD + _T,), jnp.float32),
            pltpu.VMEM((_T,), jnp.float32),
            pltpu.VMEM((_T,), jnp.float32),
            pltpu.VMEM((_T,), jnp.float32),
        ],
    )(_flanger_body)
    return fn(x, mod_sig)


def kernel(x, mod_sig):
    return _flanger(x, mod_sig)
